# baseline (speedup 1.0000x reference)
import functools

import jax
import jax.numpy as jnp
from jax import lax
from jax.experimental import pallas as pl
from jax.experimental.pallas import tpu as pltpu
from jax.experimental.pallas import tpu_sc as plsc

T = 4
B = 4096
L = 50
D = 64
Z = 100000
NC = 2
NS = 16
NW = NC * NS
SPT = B // NW
CB = 16
CPT = SPT // CB
IDS = CB * L
KR = 8
KC = IDS // KR

_mesh = plsc.VectorSubcoreMesh(core_axis_name="c", subcore_axis_name="s")


@functools.partial(
    pl.kernel,
    out_type=jax.ShapeDtypeStruct((B // 8, 8, 2, 128), jnp.float32),
    mesh=_mesh,
    scratch_types=[
        pltpu.VMEM((KR, KC), jnp.int32),
        pltpu.VMEM((KR, KC), jnp.int32),
        pltpu.VMEM((KR, KC), jnp.int32),
        pltpu.VMEM((KR, KC), jnp.int32),
        pltpu.VMEM((IDS, D), jnp.float32),
        pltpu.VMEM((IDS, D), jnp.float32),
        pltpu.VMEM((2, 8, 2, 128), jnp.float32),
        pltpu.SemaphoreType.DMA,
        pltpu.SemaphoreType.DMA,
    ],
    compiler_params=pltpu.CompilerParams(use_tc_tiling_on_sc=False),
)
def _emb(feat_hbm, t0, t1, t2, t3, out_hbm,
         fidx0, fidx1, fidx2, fidx3, rows0, rows1, outb_v, sem0, sem1):
    cid = lax.axis_index("c")
    sid = lax.axis_index("s")
    wid = sid * NC + cid
    tables = [t0, t1, t2, t3]
    fidx = [fidx0, fidx1, fidx2, fidx3]
    rows = [rows0, rows1]
    sems = [sem0, sem1]

    def vmod(v):
        t = v & 31
        v5 = lax.shift_right_logical(v, 5)
        a = lax.shift_right_logical(v5, 13)
        b = v5 & 8191
        w = a * 1942 + b
        q = (w.astype(jnp.float32) * (1.0 / 3125.0)).astype(jnp.int32)
        r = w - q * 3125
        r = jnp.where(r < 0, r + 3125, r)
        r = jnp.where(r >= 3125, r - 3125, r)
        return lax.shift_left(lax.shift_left(r, 5) | t, 1)

    def stage_ids(f, c):
        fx = fidx[f]
        pltpu.sync_copy(feat_hbm.at[f, wid * CPT + c], fx)

        def mod_body(k, _):
            for o in (0, 16, 32, 48, 64, 80):
                fx[k, pl.ds(o, 16)] = vmod(fx[k, pl.ds(o, 16)])
            v = fx[k, pl.ds(84, 16)]
            lane = lax.iota(jnp.int32, 16)
            fx[k, pl.ds(84, 16)] = jnp.where(lane >= 12, vmod(v), v)
            return 0

        lax.fori_loop(0, KR, mod_body, 0)

    def fire(f, buf):
        tab = tables[f]
        fx = fidx[f]
        rw = rows[buf]
        sm = sems[buf]

        def gat_body(k, _):
            pltpu.make_async_copy(
                tab.at[fx.at[k]], rw.at[pl.ds(k * KC, KC)], sm
            ).start()
            return 0

        lax.fori_loop(0, KR, gat_body, 0)

    def pool(f, c, buf):
        rw = rows[buf]
        pltpu.make_async_copy(
            tables[f].at[pl.ds(0, IDS)], rw, sems[buf]
        ).wait()

        def bag_body(j, _):
            r0 = j * L
            accs = tuple(rw[r0, pl.ds(d * 16, 16)] for d in range(4))

            def l_body(i, accs):
                base = r0 + 1 + i * 7
                for u in range(7):
                    r = base + u
                    accs = tuple(
                        accs[d] + rw[r, pl.ds(d * 16, 16)] for d in range(4)
                    )
                return accs

            accs = lax.fori_loop(0, 7, l_body, accs)
            jb = lax.shift_right_logical(j, 3)
            jr = j & 7
            for d in range(4):
                col = f * D + d * 16
                outb_v[jb, jr, col // 128, pl.ds(col % 128, 16)] = accs[d]
            return 0

        lax.fori_loop(0, CB, bag_body, 0)

    def flush(c):
        rb0 = wid * (SPT // 8) + c * (CB // 8)
        pltpu.sync_copy(outb_v, out_hbm.at[pl.ds(rb0, CB // 8)])

    stage_ids(0, 0)
    stage_ids(1, 0)
    fire(0, 0)

    def chunk_body(c, _):
        for f in range(T):
            f1 = (f + 1) % T
            fire(f1, (f + 1) % 2)
            f2 = (f + 2) % T
            stage_ids(f2, c + (f + 2) // T)
            pool(f, c, f % 2)
        flush(c)
        return 0

    lax.fori_loop(0, CPT - 1, chunk_body, 0)

    c = CPT - 1
    for f in range(T):
        if f + 1 < T:
            fire(f + 1, (f + 1) % 2)
        if f + 2 < T:
            stage_ids(f + 2, c)
        pool(f, c, f % 2)
    flush(c)


def kernel(features, table_0, table_1, table_2, table_3):
    feat4 = features.reshape(T, NW * CPT, KR, KC)
    tabs = [
        jnp.pad(t, ((0, 0), (0, D))).reshape(2 * Z, D)
        for t in (table_0, table_1, table_2, table_3)
    ]
    out4 = _emb(feat4, *tabs)
    return out4.reshape(B, T * D)
